# baseline (speedup 1.0000x reference)
import functools

import numpy as np
import jax
import jax.numpy as jnp
from jax import lax
from jax.experimental import pallas as pl
from jax.experimental.pallas import tpu as pltpu
from jax.experimental.pallas import tpu_sc as plsc

_SPATIAL = ((96, 96), (48, 48), (24, 24), (12, 12))
_STARTS = (0, 9216, 11520, 12096)
_LEN_IN = 12240
_DM = 256
_H = 8
_L = 4
_P = 4
_HD = 32
_B = 2
_LQ = _LEN_IN

_QH = _B * _LQ * _H
_NW = 32
_BQ = _B * _LQ
_BQ_PER_W = _BQ // _NW
_SBQ = 3
_NSUPER = _BQ_PER_W // _SBQ
_ROWS = _SBQ * 512
_GCH = 128
_NG = _ROWS // _GCH


def _proj_bf16_body(x_ref, w_ref, b_ref, o_ref):
    o_ref[...] = (
        jnp.dot(x_ref[...], w_ref[...], preferred_element_type=jnp.float32,
                precision=lax.Precision.HIGHEST)
        + b_ref[...]
    ).astype(jnp.bfloat16)


def _project_bf16(x, w, b, rb):
    m, k = x.shape
    n = w.shape[1]
    return pl.pallas_call(
        _proj_bf16_body,
        grid=(m // rb,),
        in_specs=[
            pl.BlockSpec((rb, k), lambda i: (i, 0)),
            pl.BlockSpec((k, n), lambda i: (0, 0)),
            pl.BlockSpec((1, n), lambda i: (0, 0)),
        ],
        out_specs=pl.BlockSpec((rb, n), lambda i: (i, 0)),
        out_shape=jax.ShapeDtypeStruct((m, n), jnp.bfloat16),
    )(x, w, b.reshape(1, n))


def _proj_body(x_ref, w_ref, b_ref, o_ref):
    o_ref[...] = (
        jnp.dot(x_ref[...], w_ref[...], preferred_element_type=jnp.float32,
                precision=lax.Precision.HIGHEST)
        + b_ref[...]
    )


def _project(x, w, b, rb):
    m, k = x.shape
    n = w.shape[1]
    return pl.pallas_call(
        _proj_body,
        grid=(m // rb,),
        in_specs=[
            pl.BlockSpec((rb, k), lambda i: (i, 0)),
            pl.BlockSpec((k, n), lambda i: (0, 0)),
            pl.BlockSpec((1, n), lambda i: (0, 0)),
        ],
        out_specs=pl.BlockSpec((rb, n), lambda i: (i, 0)),
        out_shape=jax.ShapeDtypeStruct((m, n), jnp.float32),
    )(x, w, b.reshape(1, n))


def _sel_by_level(l_col, vals):
    out = jnp.full(l_col.shape, float(vals[3]), jnp.float32)
    for lv in (2, 1, 0):
        out = jnp.where(l_col == lv, float(vals[lv]), out)
    return out


def _samp_body(q_ref, rp_ref, w_ref, b_ref, iw_ref):
    nb = pl.program_id(0) // (_LQ // q_ref.shape[0])
    y = (
        jnp.dot(q_ref[...], w_ref[...], preferred_element_type=jnp.float32,
                precision=lax.Precision.HIGHEST)
        + b_ref[...]
    )
    off = y[:, :256]
    logits = y[:, 256:384]

    m = jnp.max(logits, axis=-1, keepdims=True)
    e = jnp.exp(logits - m)
    k1 = lax.broadcasted_iota(jnp.int32, (128, 128), 0)
    k2 = lax.broadcasted_iota(jnp.int32, (128, 128), 1)
    blk = ((k1 >> 4) == (k2 >> 4)).astype(jnp.float32)
    s = jnp.dot(e, blk, preferred_element_type=jnp.float32,
                precision=lax.Precision.HIGHEST)
    aw = e / s

    j = lax.broadcasted_iota(jnp.int32, (1, 256), 1)
    l_col = (j >> 3) & 3
    xy = j & 1
    h_col = (j >> 5).astype(jnp.float32)
    wls = [s_[1] for s_ in _SPATIAL]
    hls = [s_[0] for s_ in _SPATIAL]
    dim = jnp.where(xy == 0, _sel_by_level(l_col, wls), _sel_by_level(l_col, hls))
    wl_e = _sel_by_level(l_col, wls)
    hl_e = _sel_by_level(l_col, hls)
    start_e = _sel_by_level(l_col, _STARTS)

    i8 = lax.broadcasted_iota(jnp.int32, (8, 256), 0)
    j8 = lax.broadcasted_iota(jnp.int32, (8, 256), 1)
    sel8 = (i8 == ((((j8 >> 3) & 3) * 2) + (j8 & 1))).astype(jnp.float32)
    rp256 = jnp.dot(rp_ref[...], sel8, preferred_element_type=jnp.float32,
                    precision=lax.Precision.HIGHEST)

    t = rp256 * dim + off - 0.5
    t0 = jnp.floor(t)
    fr = t - t0

    a1 = lax.broadcasted_iota(jnp.int32, (256, 256), 0)
    a2 = lax.broadcasted_iota(jnp.int32, (256, 256), 1)
    perm = (a2 == (a1 ^ 1)).astype(jnp.float32)
    t0_sw = jnp.dot(t0, perm, preferred_element_type=jnp.float32,
                    precision=lax.Precision.HIGHEST)
    fr_sw = jnp.dot(fr, perm, preferred_element_type=jnp.float32,
                    precision=lax.Precision.HIGHEST)

    kk = lax.broadcasted_iota(jnp.int32, (128, 256), 0)
    jj = lax.broadcasted_iota(jnp.int32, (128, 256), 1)
    awe = ((kk == (jj >> 1)) & ((jj & 1) == 0)).astype(jnp.float32)
    aw256 = jnp.dot(aw, awe, preferred_element_type=jnp.float32,
                    precision=lax.Precision.HIGHEST)

    jc = lax.broadcasted_iota(jnp.int32, (256, 128), 0)
    kc = lax.broadcasted_iota(jnp.int32, (256, 128), 1)
    comp = (jc == (kc * 2)).astype(jnp.float32)

    base = (nb * (_LEN_IN * _H)).astype(jnp.float32)
    for c, (dx, dy) in enumerate(((0, 0), (1, 0), (0, 1), (1, 1))):
        xi = t0 + dx
        yi = t0_sw + dy
        valid = ((xi >= 0) & (xi <= wl_e - 1) & (yi >= 0) & (yi <= hl_e - 1))
        xc = jnp.clip(xi, 0.0, wl_e - 1)
        yc = jnp.clip(yi, 0.0, hl_e - 1)
        gidx = base + (yc * wl_e + xc + start_e) * _H + h_col
        bwx = fr if dx == 1 else (1.0 - fr)
        bwy = fr_sw if dy == 1 else (1.0 - fr_sw)
        wgt = aw256 * bwx * bwy * valid.astype(jnp.float32)
        gidx_c = jnp.dot(gidx, comp, preferred_element_type=jnp.float32,
                         precision=lax.Precision.HIGHEST)
        wgt_c = jnp.dot(wgt, comp, preferred_element_type=jnp.float32,
                        precision=lax.Precision.HIGHEST)
        iw_ref[:, c, :] = (gidx_c + 0.5).astype(jnp.int32)
        iw_ref[:, 4 + c, :] = lax.bitcast_convert_type(wgt_c, jnp.int32)


def _sampling_params(query, rp8, w_sa, b_sa, rb):
    bq = query.shape[0]
    return pl.pallas_call(
        _samp_body,
        grid=(bq // rb,),
        in_specs=[
            pl.BlockSpec((rb, _DM), lambda i: (i, 0)),
            pl.BlockSpec((rb, 8), lambda i: (i, 0)),
            pl.BlockSpec((_DM, 384), lambda i: (0, 0)),
            pl.BlockSpec((1, 384), lambda i: (0, 0)),
        ],
        out_specs=pl.BlockSpec((rb, 8, 128), lambda i: (i, 0, 0)),
        out_shape=jax.ShapeDtypeStruct((bq, 8, 128), jnp.int32),
    )(query, rp8, w_sa, b_sa.reshape(1, 384))


def _lane_bcast(v, j):
    idx = jnp.full((16, 1), j, jnp.int32)
    dnums = lax.GatherDimensionNumbers(
        offset_dims=(), collapsed_slice_dims=(0,), start_index_map=(0,))
    return lax.gather(v, idx, dnums, (1,),
                      mode=lax.GatherScatterMode.PROMISE_IN_BOUNDS)


_NS = _NSUPER
_IWW = _SBQ * 1024
_WW = 16


def _sc_gather(table, iw):
    mesh = plsc.VectorSubcoreMesh(core_axis_name="c", subcore_axis_name="s",
                                  num_cores=2, num_subcores=16)

    @functools.partial(
        pl.kernel,
        out_type=jax.ShapeDtypeStruct((_QH, _HD), jnp.float32),
        mesh=mesh,
        scratch_types=[
            pltpu.VMEM((_IWW,), jnp.int32),
            pltpu.VMEM((_IWW,), jnp.int32),
            pltpu.VMEM((_ROWS, _WW), jnp.float32),
            pltpu.VMEM((_ROWS, _WW), jnp.float32),
            pltpu.VMEM((_SBQ * _H, _HD), jnp.float32),
            pltpu.VMEM((_SBQ * _H, _HD), jnp.float32),
            pltpu.SemaphoreType.DMA,
            pltpu.SemaphoreType.DMA,
            pltpu.SemaphoreType.DMA,
            pltpu.SemaphoreType.DMA,
            pltpu.SemaphoreType.DMA,
            pltpu.SemaphoreType.DMA,
        ],
        compiler_params=pltpu.CompilerParams(use_tc_tiling_on_sc=False, needs_layout_passes=False),
    )
    def k(table_hbm, iw_hbm, out_hbm, iw0, iw1, rows0, rows1, ob0, ob1,
          semi0, semi1, semg0, semg1, semo0, semo1):
        iwb = (iw0, iw1)
        rowsb = (rows0, rows1)
        outb = (ob0, ob1)
        semi = (semi0, semi1)
        semg = (semg0, semg1)
        semo = (semo0, semo1)
        wid = lax.axis_index("s") * 2 + lax.axis_index("c")
        bq_base = wid * _BQ_PER_W

        def fire_iw(m, p):
            return pltpu.async_copy(
                iw_hbm.at[pl.ds((bq_base + m * _SBQ) * 1024, _IWW)],
                iwb[p], semi[p])

        def wait_iw(p):
            pltpu.make_async_copy(
                iw_hbm.at[pl.ds(0, _IWW)], iwb[p], semi[p]).wait()

        def fire_g(p):
            for bql in range(_SBQ):
                for c in range(4):
                    pltpu.async_copy(
                        table_hbm.at[iwb[p].at[pl.ds(bql * 1024 + c * 128, 128)]],
                        rowsb[p].at[pl.ds((bql * 4 + c) * 128, 128), :],
                        semg[p])

        def wait_g(p):
            for j in range(_NG):
                pltpu.make_async_copy(
                    table_hbm.at[pl.ds(0, 128), :],
                    rowsb[p].at[pl.ds(j * 128, 128), :],
                    semg[p]).wait()

        def fire_out(m, p):
            pltpu.async_copy(
                outb[p],
                out_hbm.at[pl.ds((bq_base + m * _SBQ) * _H, _SBQ * _H), :],
                semo[p])

        def wait_out(p):
            pltpu.make_async_copy(
                outb[p], out_hbm.at[pl.ds(0, _SBQ * _H), :], semo[p]).wait()

        def compute(p):
            for bql in range(_SBQ):
                def h_body(h, carry2, bql=bql):
                    a0 = jnp.zeros((16,), jnp.float32)
                    a1 = jnp.zeros((16,), jnp.float32)
                    for c in range(4):
                        wv = plsc.bitcast(
                            iwb[p][pl.ds(bql * 1024 + (4 + c) * 128 + h * 16, 16)],
                            jnp.float32)
                        re0 = (bql * 4 + c) * 128 + h * 16
                        for lp in range(16):
                            wj = _lane_bcast(wv, lp)
                            wordv = rowsb[p][re0 + lp, pl.ds(0, _WW)]
                            lo, hi = plsc.unpack(
                                plsc.bitcast(wordv, jnp.bfloat16),
                                format=plsc.PackFormat.INTERLEAVED)
                            a0 = a0 + wj * lo
                            a1 = a1 + wj * hi
                    outb[p][bql * _H + h, pl.ds(0, 16)] = a0
                    outb[p][bql * _H + h, pl.ds(16, 16)] = a1
                    return carry2

                lax.fori_loop(0, _H, h_body, 0)

        def super_body(m, carry):
            fire_iw(m, 0).wait()
            fire_g(0)
            wait_g(0)
            compute(0)
            fire_out(m, 0)
            wait_out(0)
            return carry

        lax.fori_loop(0, _NS, super_body, 0)

    return k(table, iw)


def kernel(query, reference_points, input_flatten, input_spatial_shapes,
           input_level_start_index, W_val, b_val, W_samp, b_samp, W_attn,
           b_attn, W_out, b_out):
    b, lq, dm = query.shape

    cp_ = np.empty((256,), np.int32)
    for h_ in range(_H):
        for w_ in range(16):
            cp_[h_ * 32 + 2 * w_] = h_ * 32 + w_
            cp_[h_ * 32 + 2 * w_ + 1] = h_ * 32 + 16 + w_
    value_bf = _project_bf16(input_flatten.reshape(b * _LEN_IN, dm),
                             W_val[:, cp_], b_val[cp_], 1440)
    table = lax.bitcast_convert_type(
        value_bf.reshape(b * _LEN_IN * _H, _WW, 2), jnp.float32)

    w_sa = jnp.concatenate([W_samp, W_attn], axis=1)
    b_sa = jnp.concatenate([b_samp, b_attn], axis=0)
    rp8 = reference_points.reshape(b * lq, 8)
    iw_all = _sampling_params(query.reshape(b * lq, dm), rp8, w_sa, b_sa, 816)

    gathered = _sc_gather(table, iw_all.reshape(-1))

    out = _project(gathered.reshape(b * lq, _DM), W_out, b_out, 1440)
    return out.reshape(b, lq, _DM)


# speedup vs baseline: 4.8144x; 4.8144x over previous
import functools

import numpy as np
import jax
import jax.numpy as jnp
from jax import lax
from jax.experimental import pallas as pl
from jax.experimental.pallas import tpu as pltpu
from jax.experimental.pallas import tpu_sc as plsc

_SPATIAL = ((96, 96), (48, 48), (24, 24), (12, 12))
_STARTS = (0, 9216, 11520, 12096)
_LEN_IN = 12240
_DM = 256
_H = 8
_L = 4
_P = 4
_HD = 32
_B = 2
_LQ = _LEN_IN

_QH = _B * _LQ * _H
_NW = 32
_BQ = _B * _LQ
_BQ_PER_W = _BQ // _NW
_SBQ = 3
_NSUPER = _BQ_PER_W // _SBQ
_ROWS = _SBQ * 512
_GCH = 128
_NG = _ROWS // _GCH


def _proj_bf16_body(x_ref, w_ref, b_ref, o_ref):
    o_ref[...] = (
        jnp.dot(x_ref[...], w_ref[...], preferred_element_type=jnp.float32,
                precision=lax.Precision.HIGHEST)
        + b_ref[...]
    ).astype(jnp.bfloat16)


def _project_bf16(x, w, b, rb):
    m, k = x.shape
    n = w.shape[1]
    return pl.pallas_call(
        _proj_bf16_body,
        grid=(m // rb,),
        in_specs=[
            pl.BlockSpec((rb, k), lambda i: (i, 0)),
            pl.BlockSpec((k, n), lambda i: (0, 0)),
            pl.BlockSpec((1, n), lambda i: (0, 0)),
        ],
        out_specs=pl.BlockSpec((rb, n), lambda i: (i, 0)),
        out_shape=jax.ShapeDtypeStruct((m, n), jnp.bfloat16),
    )(x, w, b.reshape(1, n))


def _proj_body(x_ref, w_ref, b_ref, o_ref):
    o_ref[...] = (
        jnp.dot(x_ref[...], w_ref[...], preferred_element_type=jnp.float32,
                precision=lax.Precision.HIGHEST)
        + b_ref[...]
    )


def _project(x, w, b, rb):
    m, k = x.shape
    n = w.shape[1]
    return pl.pallas_call(
        _proj_body,
        grid=(m // rb,),
        in_specs=[
            pl.BlockSpec((rb, k), lambda i: (i, 0)),
            pl.BlockSpec((k, n), lambda i: (0, 0)),
            pl.BlockSpec((1, n), lambda i: (0, 0)),
        ],
        out_specs=pl.BlockSpec((rb, n), lambda i: (i, 0)),
        out_shape=jax.ShapeDtypeStruct((m, n), jnp.float32),
    )(x, w, b.reshape(1, n))


def _sel_by_level(l_col, vals):
    out = jnp.full(l_col.shape, float(vals[3]), jnp.float32)
    for lv in (2, 1, 0):
        out = jnp.where(l_col == lv, float(vals[lv]), out)
    return out


def _samp_body(q_ref, rp_ref, w_ref, b_ref, idx_ref, wgt_ref):
    nb = pl.program_id(0) // (_LQ // q_ref.shape[0])
    y = (
        jnp.dot(q_ref[...], w_ref[...], preferred_element_type=jnp.float32,
                precision=lax.Precision.HIGHEST)
        + b_ref[...]
    )
    off = y[:, :256]
    logits = y[:, 256:384]

    m = jnp.max(logits, axis=-1, keepdims=True)
    e = jnp.exp(logits - m)
    k1 = lax.broadcasted_iota(jnp.int32, (128, 128), 0)
    k2 = lax.broadcasted_iota(jnp.int32, (128, 128), 1)
    blk = ((k1 >> 4) == (k2 >> 4)).astype(jnp.float32)
    s = jnp.dot(e, blk, preferred_element_type=jnp.float32,
                precision=lax.Precision.HIGHEST)
    aw = e / s

    j = lax.broadcasted_iota(jnp.int32, (1, 256), 1)
    l_col = (j >> 3) & 3
    xy = j & 1
    h_col = (j >> 5).astype(jnp.float32)
    wls = [s_[1] for s_ in _SPATIAL]
    hls = [s_[0] for s_ in _SPATIAL]
    dim = jnp.where(xy == 0, _sel_by_level(l_col, wls), _sel_by_level(l_col, hls))
    wl_e = _sel_by_level(l_col, wls)
    hl_e = _sel_by_level(l_col, hls)
    start_e = _sel_by_level(l_col, _STARTS)

    i8 = lax.broadcasted_iota(jnp.int32, (8, 256), 0)
    j8 = lax.broadcasted_iota(jnp.int32, (8, 256), 1)
    sel8 = (i8 == ((((j8 >> 3) & 3) * 2) + (j8 & 1))).astype(jnp.float32)
    rp256 = jnp.dot(rp_ref[...], sel8, preferred_element_type=jnp.float32,
                    precision=lax.Precision.HIGHEST)

    t = rp256 * dim + off - 0.5
    t0 = jnp.floor(t)
    fr = t - t0

    a1 = lax.broadcasted_iota(jnp.int32, (256, 256), 0)
    a2 = lax.broadcasted_iota(jnp.int32, (256, 256), 1)
    perm = (a2 == (a1 ^ 1)).astype(jnp.float32)
    t0_sw = jnp.dot(t0, perm, preferred_element_type=jnp.float32,
                    precision=lax.Precision.HIGHEST)
    fr_sw = jnp.dot(fr, perm, preferred_element_type=jnp.float32,
                    precision=lax.Precision.HIGHEST)

    kk = lax.broadcasted_iota(jnp.int32, (128, 256), 0)
    jj = lax.broadcasted_iota(jnp.int32, (128, 256), 1)
    awe = ((kk == (jj >> 1)) & ((jj & 1) == 0)).astype(jnp.float32)
    aw256 = jnp.dot(aw, awe, preferred_element_type=jnp.float32,
                    precision=lax.Precision.HIGHEST)

    jc = lax.broadcasted_iota(jnp.int32, (256, 128), 0)
    kc = lax.broadcasted_iota(jnp.int32, (256, 128), 1)
    comp = (jc == (kc * 2)).astype(jnp.float32)

    base = (nb * (_LEN_IN * _H)).astype(jnp.float32)
    for c, (dx, dy) in enumerate(((0, 0), (1, 0), (0, 1), (1, 1))):
        xi = t0 + dx
        yi = t0_sw + dy
        valid = ((xi >= 0) & (xi <= wl_e - 1) & (yi >= 0) & (yi <= hl_e - 1))
        xc = jnp.clip(xi, 0.0, wl_e - 1)
        yc = jnp.clip(yi, 0.0, hl_e - 1)
        gidx = base + (yc * wl_e + xc + start_e) * _H + h_col
        bwx = fr if dx == 1 else (1.0 - fr)
        bwy = fr_sw if dy == 1 else (1.0 - fr_sw)
        wgt = aw256 * bwx * bwy * valid.astype(jnp.float32)
        gidx_c = jnp.dot(gidx, comp, preferred_element_type=jnp.float32,
                         precision=lax.Precision.HIGHEST)
        wgt_c = jnp.dot(wgt, comp, preferred_element_type=jnp.float32,
                        precision=lax.Precision.HIGHEST)
        idx_ref[:, c, :] = (gidx_c + 0.5).astype(jnp.int32)
        wgt_ref[:, c, :] = wgt_c


def _sampling_params(query, rp8, w_sa, b_sa, rb):
    bq = query.shape[0]
    return pl.pallas_call(
        _samp_body,
        grid=(bq // rb,),
        in_specs=[
            pl.BlockSpec((rb, _DM), lambda i: (i, 0)),
            pl.BlockSpec((rb, 8), lambda i: (i, 0)),
            pl.BlockSpec((_DM, 384), lambda i: (0, 0)),
            pl.BlockSpec((1, 384), lambda i: (0, 0)),
        ],
        out_specs=(
            pl.BlockSpec((rb, 4, 128), lambda i: (i, 0, 0)),
            pl.BlockSpec((rb, 4, 128), lambda i: (i, 0, 0)),
        ),
        out_shape=(
            jax.ShapeDtypeStruct((bq, 4, 128), jnp.int32),
            jax.ShapeDtypeStruct((bq, 4, 128), jnp.float32),
        ),
    )(query, rp8, w_sa, b_sa.reshape(1, 384))


def _lane_bcast(v, j):
    idx = jnp.full((16, 1), j, jnp.int32)
    dnums = lax.GatherDimensionNumbers(
        offset_dims=(), collapsed_slice_dims=(0,), start_index_map=(0,))
    return lax.gather(v, idx, dnums, (1,),
                      mode=lax.GatherScatterMode.PROMISE_IN_BOUNDS)


_NS = _NSUPER


def _sc_gather(table, idx, wgt):
    mesh = plsc.VectorSubcoreMesh(core_axis_name="c", subcore_axis_name="s",
                                  num_cores=2, num_subcores=16)

    @functools.partial(
        pl.kernel,
        out_type=jax.ShapeDtypeStruct((_QH, _HD), jnp.float32),
        mesh=mesh,
        scratch_types=[
            pltpu.VMEM((_ROWS,), jnp.int32),
            pltpu.VMEM((_ROWS,), jnp.int32),
            pltpu.VMEM((_ROWS,), jnp.float32),
            pltpu.VMEM((_ROWS,), jnp.float32),
            pltpu.VMEM((_ROWS, _HD), jnp.float32),
            pltpu.VMEM((_ROWS, _HD), jnp.float32),
            pltpu.VMEM((_SBQ * _H, _HD), jnp.float32),
            pltpu.VMEM((_SBQ * _H, _HD), jnp.float32),
            pltpu.SemaphoreType.DMA,
            pltpu.SemaphoreType.DMA,
            pltpu.SemaphoreType.DMA,
            pltpu.SemaphoreType.DMA,
            pltpu.SemaphoreType.DMA,
            pltpu.SemaphoreType.DMA,
        ],
        compiler_params=pltpu.CompilerParams(use_tc_tiling_on_sc=False),
    )
    def k(table_hbm, idx_hbm, wgt_hbm, out_hbm, ix0, ix1, wg0, wg1,
          rows0, rows1, ob0, ob1, semi0, semi1, semg0, semg1, semo0, semo1):
        ixb = (ix0, ix1)
        wgb = (wg0, wg1)
        rowsb = (rows0, rows1)
        outb = (ob0, ob1)
        semi = (semi0, semi1)
        semg = (semg0, semg1)
        semo = (semo0, semo1)
        wid = lax.axis_index("s") * 2 + lax.axis_index("c")
        bq_base = wid * _BQ_PER_W

        def fire_iw(m, p):
            e0 = (bq_base + m * _SBQ) * 512
            pltpu.async_copy(idx_hbm.at[pl.ds(e0, _ROWS)], ixb[p], semi[p])
            pltpu.async_copy(wgt_hbm.at[pl.ds(e0, _ROWS)], wgb[p], semi[p])

        def wait_iw(p):
            pltpu.make_async_copy(
                idx_hbm.at[pl.ds(0, _ROWS)], ixb[p], semi[p]).wait()
            pltpu.make_async_copy(
                wgt_hbm.at[pl.ds(0, _ROWS)], wgb[p], semi[p]).wait()

        def fire_g(p):
            for j in range(_NG):
                pltpu.async_copy(
                    table_hbm.at[ixb[p].at[pl.ds(j * _GCH, _GCH)]],
                    rowsb[p].at[pl.ds(j * _GCH, _GCH), :],
                    semg[p])

        def wait_g(p):
            for j in range(_NG):
                pltpu.make_async_copy(
                    table_hbm.at[pl.ds(0, _GCH), :],
                    rowsb[p].at[pl.ds(j * _GCH, _GCH), :],
                    semg[p]).wait()

        def fire_out(m, p):
            pltpu.async_copy(
                outb[p],
                out_hbm.at[pl.ds((bq_base + m * _SBQ) * _H, _SBQ * _H), :],
                semo[p])

        def wait_out(p):
            pltpu.make_async_copy(
                outb[p], out_hbm.at[pl.ds(0, _SBQ * _H), :], semo[p]).wait()

        def compute(p):
            for bql in range(_SBQ):
                def h_body(h, carry2, bql=bql):
                    a0 = jnp.zeros((16,), jnp.float32)
                    a1 = jnp.zeros((16,), jnp.float32)
                    for c in range(4):
                        base_e = (bql * 4 + c) * 128 + h * 16
                        wvec = wgb[p][pl.ds(base_e, 16)]
                        for lp in range(16):
                            wj = _lane_bcast(wvec, lp)
                            r0 = rowsb[p][base_e + lp, pl.ds(0, 16)]
                            r1 = rowsb[p][base_e + lp, pl.ds(16, 16)]
                            a0 = a0 + wj * r0
                            a1 = a1 + wj * r1
                    outb[p][bql * _H + h, pl.ds(0, 16)] = a0
                    outb[p][bql * _H + h, pl.ds(16, 16)] = a1
                    return carry2

                lax.fori_loop(0, _H, h_body, 0)

        fire_iw(0, 0)
        wait_iw(0)
        fire_g(0)
        fire_iw(1, 1)

        def pair_body(g, carry):
            for p in (0, 1):
                m = 2 * g + p

                @pl.when(m < _NS - 1)
                def _():
                    wait_iw(1 - p)
                    fire_g(1 - p)

                @pl.when(m < _NS)
                def _():
                    @pl.when(m >= 2)
                    def _():
                        wait_out(p)
                    wait_g(p)
                    compute(p)
                    fire_out(m, p)

                @pl.when(m < _NS - 2)
                def _():
                    fire_iw(m + 2, p)

            return carry

        lax.fori_loop(0, (_NS + 1) // 2, pair_body, 0)
        wait_out(1)
        wait_out(0)

    return k(table, idx, wgt)


def kernel(query, reference_points, input_flatten, input_spatial_shapes,
           input_level_start_index, W_val, b_val, W_samp, b_samp, W_attn,
           b_attn, W_out, b_out):
    b, lq, dm = query.shape

    value = _project(input_flatten.reshape(b * _LEN_IN, dm), W_val, b_val, 1440)
    table = value.reshape(b * _LEN_IN * _H, _HD)

    w_sa = jnp.concatenate([W_samp, W_attn], axis=1)
    b_sa = jnp.concatenate([b_samp, b_attn], axis=0)
    rp8 = reference_points.reshape(b * lq, 8)
    idx_all, wgt_all = _sampling_params(query.reshape(b * lq, dm), rp8,
                                        w_sa, b_sa, 816)

    gathered = _sc_gather(table, idx_all.reshape(-1), wgt_all.reshape(-1))

    out = _project(gathered.reshape(b * lq, _DM), W_out, b_out, 1440)
    return out.reshape(b, lq, _DM)
